# baseline (speedup 1.0000x reference)
import functools

import jax
import jax.numpy as jnp
from jax.experimental import pallas as pl
from jax.experimental.pallas import tpu as pltpu

N = 50000
E = 800000
DIM = 64
B = 1024
STEPS = 12

_ROWS = 1000


def _mm_kernel(x_ref, w_ref, b_ref, o_ref, *, leaky):
    acc = jnp.dot(x_ref[...], w_ref[...], preferred_element_type=jnp.float32)
    acc = acc + b_ref[...]
    if leaky:
        acc = jnp.where(acc >= 0, acc, 0.01 * acc)
    o_ref[...] = acc


def _mm(x, W, b, leaky=False):
    n, k = x.shape
    kout = W.shape[1]
    rows = _ROWS if n % _ROWS == 0 else n
    grid = (n // rows,)
    return pl.pallas_call(
        functools.partial(_mm_kernel, leaky=leaky),
        grid=grid,
        in_specs=[
            pl.BlockSpec((rows, k), lambda i: (i, 0)),
            pl.BlockSpec((k, kout), lambda i: (0, 0)),
            pl.BlockSpec((kout,), lambda i: (0,)),
        ],
        out_specs=pl.BlockSpec((rows, kout), lambda i: (i, 0)),
        out_shape=jax.ShapeDtypeStruct((n, kout), jnp.float32),
    )(x, W, b)


def kernel(x, edge_index, edge_attr, batch, W0, b0, Wq, bq, Wk, bk, Wv, bv,
           We, be, Ws, bs, W3a, b3a, W3b, b3b, W3c, b3c):
    act = lambda t: jax.nn.leaky_relu(t, 0.01)
    src = edge_index[0]
    dst = edge_index[1]
    out = _mm(x, W0, b0, leaky=True)
    e = edge_attr @ We + be
    Wqkvs = jnp.concatenate([Wq, Wk, Wv, Ws], axis=1)
    bqkvs = jnp.concatenate([bq, bk, bv, bs], axis=0)
    for _ in range(STEPS):
        qkvs = _mm(out, Wqkvs, bqkvs)
        q = qkvs[:, 0:DIM]
        kk = qkvs[:, DIM:2 * DIM]
        vv = qkvs[:, 2 * DIM:3 * DIM]
        ss = qkvs[:, 3 * DIM:4 * DIM]
        kj = kk[src] + e
        vj = vv[src] + e
        logits = jnp.sum(q[dst] * kj, axis=-1) / jnp.sqrt(jnp.float32(DIM))
        m = jax.ops.segment_max(logits, dst, num_segments=N)
        m = jnp.where(jnp.isfinite(m), m, 0.0)
        ex = jnp.exp(logits - m[dst])
        den = jax.ops.segment_sum(ex, dst, num_segments=N)
        alpha = ex / (den[dst] + 1e-16)
        agg = jax.ops.segment_sum(alpha[:, None] * vj, dst, num_segments=N)
        out = act(agg + ss)
    sums = jax.ops.segment_sum(out, batch, num_segments=B)
    cnt = jax.ops.segment_sum(jnp.ones((N,), jnp.float32), batch, num_segments=B)
    g = sums / jnp.clip(cnt, 1.0, None)[:, None]
    h = act(g @ W3a + b3a)
    h = act(h @ W3b + b3b)
    per_mol_out = h @ W3c + b3c
    return per_mol_out


# speedup vs baseline: 3.1828x; 3.1828x over previous
import functools

import jax
import jax.numpy as jnp
from jax import lax
from jax.experimental import pallas as pl
from jax.experimental.pallas import tpu as pltpu
from jax.experimental.pallas import tpu_sc as plsc

N = 50000
E = 800000
DIM = 64
HALF = 32
B = 1024
STEPS = 12

NC = 2
NS = 16
NW = NC * NS
L = 16

SEG = 3136
N2 = SEG * NS
CH = 128
NCHUNK = E // CH
CPT = (NCHUNK + NW - 1) // NW

EROWS = 2000
EBLK = E // EROWS
NROWS = 1000

_mesh = plsc.VectorSubcoreMesh(core_axis_name="c", subcore_axis_name="s")


def _mm_kernel(x_ref, w_ref, b_ref, o_ref, *, leaky, pad_ones):
    acc = jnp.dot(x_ref[...], w_ref[...], preferred_element_type=jnp.float32)
    acc = acc + b_ref[...]
    if leaky:
        acc = jnp.where(acc >= 0, acc, 0.01 * acc)
    if pad_ones:
        acc = jnp.concatenate(
            [acc, jnp.ones(acc.shape, jnp.float32)], axis=1)
    o_ref[...] = acc


def _mm(x, W, b, leaky=False, rows=NROWS, pad_ones=False):
    n, k = x.shape
    kout = W.shape[1]
    if n % rows != 0:
        rows = n
    oc = 2 * kout if pad_ones else kout
    return pl.pallas_call(
        functools.partial(_mm_kernel, leaky=leaky, pad_ones=pad_ones),
        grid=(n // rows,),
        in_specs=[
            pl.BlockSpec((rows, k), lambda i: (i, 0)),
            pl.BlockSpec((k, kout), lambda i: (0, 0)),
            pl.BlockSpec((kout,), lambda i: (0,)),
        ],
        out_specs=pl.BlockSpec((rows, oc), lambda i: (i, 0)),
        out_shape=jax.ShapeDtypeStruct((n, oc), jnp.float32),
    )(x, W, b)


def _qkvs_kernel(x_ref, w_ref, b_ref, t1_ref, t2_ref, s_ref):
    acc = jnp.dot(x_ref[:, 0:DIM], w_ref[...],
                  preferred_element_type=jnp.float32)
    acc = acc + b_ref[...]
    q = acc[:, 0:DIM]
    t1_ref[...] = acc[:, DIM:3 * DIM]
    t2_ref[...] = jnp.concatenate([q, q], axis=1)
    s_ref[...] = acc[:, 3 * DIM:4 * DIM]


def _qkvs(out, Wc, bc):
    o2 = jax.ShapeDtypeStruct((N, 2 * DIM), jnp.float32)
    return pl.pallas_call(
        _qkvs_kernel,
        grid=(N // NROWS,),
        in_specs=[
            pl.BlockSpec((NROWS, 2 * DIM), lambda i: (i, 0)),
            pl.BlockSpec((DIM, 4 * DIM), lambda i: (0, 0)),
            pl.BlockSpec((4 * DIM,), lambda i: (0,)),
        ],
        out_specs=[
            pl.BlockSpec((NROWS, 2 * DIM), lambda i: (i, 0)),
            pl.BlockSpec((NROWS, 2 * DIM), lambda i: (i, 0)),
            pl.BlockSpec((NROWS, DIM), lambda i: (i, 0)),
        ],
        out_shape=[o2, o2, jax.ShapeDtypeStruct((N, DIM), jnp.float32)],
    )(out, Wc, bc)


def _logits_kernel(qd_ref, kv_ref, e_ref, l_ref, m_ref):
    i = pl.program_id(0)
    kj = kv_ref[:, 0:DIM] + e_ref[...]
    lg = jnp.sum(qd_ref[:, 0:DIM] * kj, axis=1) * 0.125
    l_ref[0, 0, :] = lg
    bm = jnp.max(lg)

    @pl.when(i == 0)
    def _():
        m_ref[0, 0] = bm

    m_ref[0, 0] = jnp.maximum(m_ref[0, 0], bm)


def _logits(qd, kvs, e):
    return pl.pallas_call(
        _logits_kernel,
        grid=(EBLK,),
        in_specs=[
            pl.BlockSpec((EROWS, 2 * DIM), lambda i: (i, 0)),
            pl.BlockSpec((EROWS, 2 * DIM), lambda i: (i, 0)),
            pl.BlockSpec((EROWS, DIM), lambda i: (i, 0)),
        ],
        out_specs=[
            pl.BlockSpec((1, 1, EROWS), lambda i: (i, 0, 0)),
            pl.BlockSpec(memory_space=pltpu.SMEM),
        ],
        out_shape=[
            jax.ShapeDtypeStruct((EBLK, 1, EROWS), jnp.float32),
            jax.ShapeDtypeStruct((1, 1), jnp.float32),
        ],
    )(qd, kvs, e)


def _w_kernel(l_ref, m_ref, kv_ref, e_ref, w_ref):
    lg = jnp.reshape(l_ref[...], (EROWS, 1))
    ex = jnp.exp(lg - m_ref[0, 0])
    wv = ex * (kv_ref[:, DIM:2 * DIM] + e_ref[...])
    w_ref[...] = jnp.concatenate(
        [wv, jnp.broadcast_to(ex, (EROWS, DIM))], axis=1)


def _wmsg(l3, M, kvs, e):
    return pl.pallas_call(
        _w_kernel,
        grid=(EBLK,),
        in_specs=[
            pl.BlockSpec((1, 1, EROWS), lambda i: (i, 0, 0)),
            pl.BlockSpec(memory_space=pltpu.SMEM),
            pl.BlockSpec((EROWS, 2 * DIM), lambda i: (i, 0)),
            pl.BlockSpec((EROWS, DIM), lambda i: (i, 0)),
        ],
        out_specs=pl.BlockSpec((EROWS, 2 * DIM), lambda i: (i, 0)),
        out_shape=jax.ShapeDtypeStruct((E, 2 * DIM), jnp.float32),
    )(l3, M, kvs, e)


def _outstep_kernel(agg_ref, ss_ref, o_ref):
    den = agg_ref[:, DIM:DIM + 1]
    t = agg_ref[:, 0:DIM] / (den + 1e-16) + ss_ref[...]
    t = jnp.where(t >= 0, t, 0.01 * t)
    o_ref[...] = jnp.concatenate(
        [t, jnp.ones((NROWS, DIM), jnp.float32)], axis=1)


def _outstep(agg, ss):
    return pl.pallas_call(
        _outstep_kernel,
        grid=(N // NROWS,),
        in_specs=[
            pl.BlockSpec((NROWS, 2 * DIM), lambda i: (i, 0)),
            pl.BlockSpec((NROWS, DIM), lambda i: (i, 0)),
        ],
        out_specs=pl.BlockSpec((NROWS, 2 * DIM), lambda i: (i, 0)),
        out_shape=jax.ShapeDtypeStruct((N, 2 * DIM), jnp.float32),
    )(agg, ss)


def _head_kernel(sp_ref, wa_ref, ba_ref, wb_ref, bb_ref, wc_ref,
                 bc_ref, o_ref):
    sp = sp_ref[0] + sp_ref[1]
    sums = sp[:, 0:DIM]
    cnt = sp[:, DIM:DIM + 1]
    g = sums / jnp.clip(cnt, 1.0, None)
    h = jnp.dot(g, wa_ref[...], preferred_element_type=jnp.float32) + ba_ref[...]
    h = jnp.where(h >= 0, h, 0.01 * h)
    h = jnp.dot(h, wb_ref[...], preferred_element_type=jnp.float32) + bb_ref[...]
    h = jnp.where(h >= 0, h, 0.01 * h)
    h = jnp.dot(h, wc_ref[...], preferred_element_type=jnp.float32) + bc_ref[...]
    o_ref[...] = h


def _head(sums_p, W3a, b3a, W3b, b3b, W3c, b3c):
    return pl.pallas_call(
        _head_kernel,
        out_shape=jax.ShapeDtypeStruct((B, 1), jnp.float32),
    )(sums_p, W3a, b3a, W3b, b3b, W3c, b3c)


def _wid():
    return lax.axis_index("s") * NC + lax.axis_index("c")


def _chunk_loop(body):
    w = _wid()

    def step(i, carry):
        g = w + i * NW

        @pl.when(g < NCHUNK)
        def _():
            body(g)

        return carry

    lax.fori_loop(0, CPT, step, 0)


@functools.partial(
    pl.kernel,
    out_type=[jax.ShapeDtypeStruct((E, 2 * DIM), jnp.float32)] * 2,
    mesh=_mesh,
    scratch_types=[
        pltpu.VMEM((CH,), jnp.int32),
        pltpu.VMEM((CH,), jnp.int32),
        pltpu.VMEM((CH, 2 * DIM), jnp.float32),
        pltpu.VMEM((CH, 2 * DIM), jnp.float32),
        pltpu.SemaphoreType.DMA,
    ],
)
def _sc_gather2(t1_hbm, t2_hbm, src_hbm, dst_hbm,
                kv_hbm, qd_hbm,
                idxs_v, idxd_v, b1_v, b2_v, sem):
    def body(g):
        base = g * CH
        pltpu.sync_copy(src_hbm.at[pl.ds(base, CH)], idxs_v)
        pltpu.sync_copy(dst_hbm.at[pl.ds(base, CH)], idxd_v)
        c1 = pltpu.async_copy(t1_hbm.at[idxs_v], b1_v, sem)
        c2 = pltpu.async_copy(t2_hbm.at[idxd_v], b2_v, sem)
        c1.wait()
        c2.wait()
        pltpu.sync_copy(b1_v, kv_hbm.at[pl.ds(base, CH)])
        pltpu.sync_copy(b2_v, qd_hbm.at[pl.ds(base, CH)])

    _chunk_loop(body)


_DW = 16
NQ = 6400
NQA = 6528
_QTR = NQA // NS
CPT2 = (NCHUNK + NS - 1) // NS


@functools.partial(
    pl.kernel,
    out_type=jax.ShapeDtypeStruct((NC, NQA, 2 * DIM), jnp.float32),
    mesh=_mesh,
    scratch_types=[
        pltpu.VMEM((CH, 2 * DIM), jnp.float32),
        pltpu.VMEM((CH,), jnp.int32),
        pltpu.VMEM((CH,), jnp.int32),
        pltpu.VMEM((L,), jnp.int32),
        pltpu.VMEM((_QTR, 2 * DIM), jnp.float32),
        pltpu.VMEM_SHARED((NQA, 2 * DIM), jnp.float32),
    ],
)
def _sc_scatter(w_hbm, dst_hbm, qb_hbm, agg_hbm,
                wbuf, ibuf, libuf, qbuf, stg, aggs):
        c = lax.axis_index("c")
        s = lax.axis_index("s")
        pltpu.sync_copy(qb_hbm, qbuf)
        base = qbuf[...][0] + c * NQ

        zv = jnp.zeros((L,), jnp.float32)

        def zs(i, carry):
            for j in range(2 * DIM // L):
                stg[i, pl.ds(j * L, L)] = zv
            return carry

        lax.fori_loop(0, _QTR, zs, 0)
        pltpu.sync_copy(stg, aggs.at[pl.ds(s * _QTR, _QTR)])
        plsc.subcore_barrier()

        def body(i, carry):
            g = s + i * NS

            @pl.when(g < NCHUNK)
            def _():
                eb = g * CH
                pltpu.sync_copy(dst_hbm.at[pl.ds(eb, CH)], ibuf)

                def lstep(k, carry2):
                    dv = ibuf[pl.ds(k * L, L)]
                    li = dv - base
                    ok = (li >= 0) & (li < NQ)
                    libuf[pl.ds(k * L, L)] = jnp.where(ok, li, NQ)
                    return carry2

                lax.fori_loop(0, CH // L, lstep, 0)
                pltpu.sync_copy(w_hbm.at[pl.ds(eb, CH)], wbuf)
                pltpu.sync_copy(wbuf, aggs.at[libuf], add=True)

            return carry

        lax.fori_loop(0, CPT2, body, 0)

        plsc.subcore_barrier()
        pltpu.sync_copy(aggs.at[pl.ds(s * _QTR, _QTR)], stg)
        pltpu.sync_copy(stg, agg_hbm.at[c, pl.ds(s * _QTR, _QTR)])


_PCH = 128
_PCHN = N // _PCH
_PTAIL = N - _PCHN * _PCH
_PCPT = (_PCHN + NW - 1) // NW
_BSEG = B // NS


@functools.partial(
    pl.kernel,
    out_type=jax.ShapeDtypeStruct((NC, B, 2 * DIM), jnp.float32),
    mesh=_mesh,
    scratch_types=[
        pltpu.VMEM((_PCH, 2 * DIM), jnp.float32),
        pltpu.VMEM((_PCH,), jnp.int32),
        pltpu.VMEM((_PTAIL, 2 * DIM), jnp.float32),
        pltpu.VMEM((_PTAIL,), jnp.int32),
        pltpu.VMEM((_BSEG, 2 * DIM), jnp.float32),
        pltpu.VMEM_SHARED((B, 2 * DIM), jnp.float32),
    ],
)
def _sc_pool(out_hbm, batch_hbm, sums_hbm,
             rbuf, ibuf, trbuf, tibuf, stgs, sums_s):
    c = lax.axis_index("c")
    s = lax.axis_index("s")
    w = _wid()

    zv = jnp.zeros((L,), jnp.float32)

    def zs(i, carry):
        for j in range(2 * DIM // L):
            stgs[i, pl.ds(j * L, L)] = zv
        return carry

    lax.fori_loop(0, _BSEG, zs, 0)

    pltpu.sync_copy(stgs, sums_s.at[pl.ds(s * _BSEG, _BSEG)])
    plsc.subcore_barrier()

    def scatter_chunk(n, rb, ib, base):
        pltpu.sync_copy(batch_hbm.at[pl.ds(base, n)], ib)
        pltpu.sync_copy(out_hbm.at[pl.ds(base, n)], rb)
        pltpu.sync_copy(rb, sums_s.at[ib], add=True)

    def step(i, carry):
        g = w + i * NW

        @pl.when(g < _PCHN)
        def _():
            scatter_chunk(_PCH, rbuf, ibuf, g * _PCH)

        return carry

    lax.fori_loop(0, _PCPT, step, 0)

    @pl.when(w == 0)
    def _():
        scatter_chunk(_PTAIL, trbuf, tibuf, _PCHN * _PCH)

    plsc.subcore_barrier()
    pltpu.sync_copy(sums_s.at[pl.ds(s * _BSEG, _BSEG)], stgs)
    pltpu.sync_copy(stgs, sums_hbm.at[c, pl.ds(s * _BSEG, _BSEG)])


def kernel(x, edge_index, edge_attr, batch, W0, b0, Wq, bq, Wk, bk, Wv, bv,
           We, be, Ws, bs, W3a, b3a, W3b, b3b, W3c, b3c):
    src = edge_index[0]
    dst = edge_index[1]

    out = _mm(x, W0, b0, leaky=True, pad_ones=True)
    e = _mm(edge_attr, We, be, rows=EROWS)
    qb = [jnp.full((L,), 2 * q * NQ, jnp.int32) for q in range(4)]
    Wc = jnp.concatenate([Wq, Wk, Wv, Ws], axis=1)
    bc = jnp.concatenate([bq, bk, bv, bs], axis=0)
    for _ in range(STEPS):
        t1, t2, ss = _qkvs(out, Wc, bc)
        kvs, qd = _sc_gather2(t1, t2, src, dst)
        l3, M = _logits(qd, kvs, e)
        w128 = _wmsg(l3, M, kvs, e)
        a0 = _sc_scatter(w128, dst, qb[0])
        dep0 = (a0[0, 0, 0] * 0.0).astype(jnp.int32)
        a1 = _sc_scatter(w128, dst, qb[1] + dep0)
        dep1 = (a1[0, 0, 0] * 0.0).astype(jnp.int32)
        a2 = _sc_scatter(w128, dst, qb[2] + dep1)
        dep2 = (a2[0, 0, 0] * 0.0).astype(jnp.int32)
        a3 = _sc_scatter(w128, dst, qb[3] + dep2)
        agg = jnp.concatenate(
            [a0[0, :NQ], a0[1, :NQ], a1[0, :NQ], a1[1, :NQ],
             a2[0, :NQ], a2[1, :NQ], a3[0, :NQ], a3[1, :NQ]], axis=0)
        out = _outstep(agg, ss)

    sums_p = _sc_pool(out, batch)
    return _head(sums_p, W3a, b3a, W3b, b3b, W3c, b3c)
